# baseline (device time: 105638 ns/iter reference)
import jax
import jax.numpy as jnp
from jax import lax
from jax.experimental import pallas as pl
from jax.experimental.pallas import tpu as pltpu

Y = 4


def kernel(x, W):
    t, d = x.shape
    _, v = W.shape

    def body(x_ref, w_ref, out_ref, gather_ref, send_sems, recv_sems):
        my_x = lax.axis_index("x")
        my_y = lax.axis_index("y")
        my_z = lax.axis_index("z")
        left = lax.rem(my_y + (Y - 1), Y)
        right = lax.rem(my_y + 1, Y)

        barrier_sem = pltpu.get_barrier_semaphore()
        for nbr in (left, right):
            pl.semaphore_signal(
                barrier_sem, inc=1,
                device_id=(my_x, nbr, my_z),
                device_id_type=pl.DeviceIdType.MESH,
            )
        pl.semaphore_wait(barrier_sem, 2)

        logits = jnp.dot(
            x_ref[...].astype(jnp.bfloat16),
            w_ref[...].astype(jnp.bfloat16),
            preferred_element_type=jnp.float32,
        )
        gather_ref[pl.ds(my_y, 1)] = logits.astype(jnp.bfloat16)[None]

        for h in range(Y - 1):
            src_slot = lax.rem(my_y + (Y - h), Y)
            rdma = pltpu.make_async_remote_copy(
                src_ref=gather_ref.at[src_slot],
                dst_ref=gather_ref.at[src_slot],
                send_sem=send_sems.at[h],
                recv_sem=recv_sems.at[h],
                device_id=(my_x, right, my_z),
                device_id_type=pl.DeviceIdType.MESH,
            )
            rdma.start()
            rdma.wait()

        m = gather_ref[0].astype(jnp.float32).max(-1, keepdims=True)
        for c in range(1, Y):
            m = jnp.maximum(
                m, gather_ref[c].astype(jnp.float32).max(-1, keepdims=True)
            )
        s = jnp.zeros((t, 1), jnp.float32)
        for c in range(Y):
            e = jnp.exp(gather_ref[c].astype(jnp.float32) - m)
            out_ref[:, c * v:(c + 1) * v] = e
            s = s + e.sum(-1, keepdims=True)
        inv = 1.0 / s
        for c in range(Y):
            out_ref[:, c * v:(c + 1) * v] = out_ref[:, c * v:(c + 1) * v] * inv

    return pl.pallas_call(
        body,
        out_shape=jax.ShapeDtypeStruct((t, Y * v), jnp.float32),
        in_specs=[
            pl.BlockSpec(memory_space=pltpu.VMEM),
            pl.BlockSpec(memory_space=pltpu.VMEM),
        ],
        out_specs=pl.BlockSpec(memory_space=pltpu.VMEM),
        scratch_shapes=[
            pltpu.VMEM((Y, t, v), jnp.bfloat16),
            pltpu.SemaphoreType.DMA((Y - 1,)),
            pltpu.SemaphoreType.DMA((Y - 1,)),
        ],
        compiler_params=pltpu.CompilerParams(collective_id=0),
    )(x, W)


# device time: 101471 ns/iter; 1.0411x vs baseline; 1.0411x over previous
import jax
import jax.numpy as jnp
from jax import lax
from jax.experimental import pallas as pl
from jax.experimental.pallas import tpu as pltpu

Y = 4


def kernel(x, W):
    t, d = x.shape
    _, v = W.shape

    def body(x_ref, w_ref, out_ref, gather_ref, send_sems, recv_sems):
        my_x = lax.axis_index("x")
        my_y = lax.axis_index("y")
        my_z = lax.axis_index("z")
        left = lax.rem(my_y + (Y - 1), Y)
        right = lax.rem(my_y + 1, Y)

        barrier_sem = pltpu.get_barrier_semaphore()
        for nbr in (left, right):
            pl.semaphore_signal(
                barrier_sem, inc=1,
                device_id=(my_x, nbr, my_z),
                device_id_type=pl.DeviceIdType.MESH,
            )
        pl.semaphore_wait(barrier_sem, 2)

        logits = jnp.dot(
            x_ref[...].astype(jnp.bfloat16),
            w_ref[...].astype(jnp.bfloat16),
            preferred_element_type=jnp.float32,
        )
        gather_ref[pl.ds(my_y, 1)] = logits.astype(jnp.bfloat16)[None]

        rdmas = []
        m = None
        s = None
        m_at = []
        slots = []

        def process(chunk_f32, slot, last):
            nonlocal m, s
            cm = chunk_f32.max(-1, keepdims=True)
            if m is None:
                mk = cm
            else:
                mk = jnp.maximum(m, cm)
            e = jnp.exp(chunk_f32 - mk)
            es = e.sum(-1, keepdims=True)
            if s is None:
                s = es
            else:
                s = s * jnp.exp(m - mk) + es
            m = mk
            if last:
                out_ref[:, pl.ds(slot * v, v)] = e * (1.0 / s)
            else:
                out_ref[:, pl.ds(slot * v, v)] = e
                m_at.append(mk)
                slots.append(slot)

        for h in range(Y - 1):
            src_slot = my_y if h == 0 else lax.rem(my_y + (Y - h), Y)
            rdma = pltpu.make_async_remote_copy(
                src_ref=gather_ref.at[src_slot],
                dst_ref=gather_ref.at[src_slot],
                send_sem=send_sems.at[h],
                recv_sem=recv_sems.at[h],
                device_id=(my_x, right, my_z),
                device_id_type=pl.DeviceIdType.MESH,
            )
            rdma.start()
            rdmas.append(rdma)
            if h == 0:
                process(logits, my_y, last=False)
            else:
                process(
                    gather_ref[pl.ds(src_slot, 1)][0].astype(jnp.float32),
                    src_slot, last=False,
                )
            rdma.wait_recv()

        last_slot = lax.rem(my_y + 1, Y)
        process(
            gather_ref[pl.ds(last_slot, 1)][0].astype(jnp.float32),
            last_slot, last=True,
        )

        inv = 1.0 / s
        for mk, slot in zip(m_at, slots):
            factor = jnp.exp(mk - m) * inv
            out_ref[:, pl.ds(slot * v, v)] = (
                out_ref[:, pl.ds(slot * v, v)] * factor
            )

        for rdma in rdmas:
            rdma.wait_send()

    return pl.pallas_call(
        body,
        out_shape=jax.ShapeDtypeStruct((t, Y * v), jnp.float32),
        in_specs=[
            pl.BlockSpec(memory_space=pltpu.VMEM),
            pl.BlockSpec(memory_space=pltpu.VMEM),
        ],
        out_specs=pl.BlockSpec(memory_space=pltpu.VMEM),
        scratch_shapes=[
            pltpu.VMEM((Y, t, v), jnp.bfloat16),
            pltpu.SemaphoreType.DMA((Y - 1,)),
            pltpu.SemaphoreType.DMA((Y - 1,)),
        ],
        compiler_params=pltpu.CompilerParams(collective_id=0),
    )(x, W)


# device time: 78135 ns/iter; 1.3520x vs baseline; 1.2987x over previous
import jax
import jax.numpy as jnp
from jax import lax
from jax.experimental import pallas as pl
from jax.experimental.pallas import tpu as pltpu

Y = 4


def kernel(x, W):
    t, d = x.shape
    _, v = W.shape
    half = t // 2
    quar = t // 4

    def body(x_ref, w_ref, out_ref, gather_ref,
             y_send_sems, y_recv_sems,
             x_send_sems, x_recv_sems,
             z_send_sems, z_recv_sems):
        my_x = lax.axis_index("x")
        my_y = lax.axis_index("y")
        my_z = lax.axis_index("z")
        left = lax.rem(my_y + (Y - 1), Y)
        right = lax.rem(my_y + 1, Y)
        px = 1 - my_x
        pz = my_z + 1 - 2 * lax.rem(my_z, 2)
        p = lax.rem(my_x + my_z, 2)
        row0 = p * half
        qrow0 = (1 - p) * half

        barrier_sem = pltpu.get_barrier_semaphore()
        neighbors = [
            (my_x, left, my_z), (my_x, right, my_z),
            (px, my_y, my_z), (my_x, my_y, pz),
        ]
        for dev in neighbors:
            pl.semaphore_signal(
                barrier_sem, inc=1, device_id=dev,
                device_id_type=pl.DeviceIdType.MESH,
            )

        logits = jnp.dot(
            x_ref[...].astype(jnp.bfloat16),
            w_ref[...].astype(jnp.bfloat16),
            preferred_element_type=jnp.float32,
        )
        gather_ref[pl.ds(my_y, 1)] = logits.astype(jnp.bfloat16)[None]

        m = None
        s = None
        m_at = []
        slots = []

        def process(chunk_f32, slot, last):
            nonlocal m, s
            cm = chunk_f32.max(-1, keepdims=True)
            mk = cm if m is None else jnp.maximum(m, cm)
            e = jnp.exp(chunk_f32 - mk)
            es = e.sum(-1, keepdims=True)
            s = es if s is None else s * jnp.exp(m - mk) + es
            m = mk
            if last:
                out_ref[:, pl.ds(slot * v, v)] = e * (1.0 / s)
            else:
                out_ref[:, pl.ds(slot * v, v)] = e
                m_at.append(mk)
                slots.append(slot)

        process(logits, my_y, last=False)

        pl.semaphore_wait(barrier_sem, len(neighbors))

        def ex_rdmas(h, slot):
            xs = pltpu.make_async_remote_copy(
                src_ref=gather_ref.at[slot, pl.ds(row0, quar)],
                dst_ref=gather_ref.at[slot, pl.ds(row0, quar)],
                send_sem=x_send_sems.at[h],
                recv_sem=x_recv_sems.at[h],
                device_id=(px, my_y, my_z),
                device_id_type=pl.DeviceIdType.MESH,
            )
            zs = pltpu.make_async_remote_copy(
                src_ref=gather_ref.at[slot, pl.ds(row0 + quar, quar)],
                dst_ref=gather_ref.at[slot, pl.ds(row0 + quar, quar)],
                send_sem=z_send_sems.at[h],
                recv_sem=z_recv_sems.at[h],
                device_id=(my_x, my_y, pz),
                device_id_type=pl.DeviceIdType.MESH,
            )
            return xs, zs

        waits = []
        pend = None
        for h in range(Y - 1):
            src_slot = my_y if h == 0 else lax.rem(my_y + (Y - h), Y)
            y_rdma = pltpu.make_async_remote_copy(
                src_ref=gather_ref.at[src_slot, pl.ds(row0, half)],
                dst_ref=gather_ref.at[src_slot, pl.ds(row0, half)],
                send_sem=y_send_sems.at[h],
                recv_sem=y_recv_sems.at[h],
                device_id=(my_x, right, my_z),
                device_id_type=pl.DeviceIdType.MESH,
            )
            y_rdma.start()
            waits.append(y_rdma)
            if pend is not None:
                xs, zs, slot = pend
                xs.wait_recv()
                zs.wait_recv()
                process(
                    gather_ref[pl.ds(slot, 1)][0].astype(jnp.float32),
                    slot, last=False,
                )
            y_rdma.wait_recv()
            recv_slot = lax.rem(my_y + (Y - 1 - h), Y)
            xs, zs = ex_rdmas(h, recv_slot)
            xs.start()
            zs.start()
            waits += [xs, zs]
            pend = (xs, zs, recv_slot)

        xs, zs, slot = pend
        xs.wait_recv()
        zs.wait_recv()
        process(
            gather_ref[pl.ds(slot, 1)][0].astype(jnp.float32),
            slot, last=True,
        )

        inv = 1.0 / s
        for mk, slot in zip(m_at, slots):
            factor = jnp.exp(mk - m) * inv
            out_ref[:, pl.ds(slot * v, v)] = (
                out_ref[:, pl.ds(slot * v, v)] * factor
            )

        for rdma in waits:
            rdma.wait_send()

        def exit_barrier(second_barrier):
            for dev in neighbors:
                pl.semaphore_signal(
                    second_barrier, inc=1, device_id=dev,
                    device_id_type=pl.DeviceIdType.MESH,
                )
            pl.semaphore_wait(second_barrier, len(neighbors))

        pl.run_scoped(exit_barrier, second_barrier=pltpu.SemaphoreType.REGULAR)

    return pl.pallas_call(
        body,
        out_shape=jax.ShapeDtypeStruct((t, Y * v), jnp.float32),
        in_specs=[
            pl.BlockSpec(memory_space=pltpu.VMEM),
            pl.BlockSpec(memory_space=pltpu.VMEM),
        ],
        out_specs=pl.BlockSpec(memory_space=pltpu.VMEM),
        scratch_shapes=[
            pltpu.VMEM((Y, t, v), jnp.bfloat16),
            pltpu.SemaphoreType.DMA((Y - 1,)),
            pltpu.SemaphoreType.DMA((Y - 1,)),
            pltpu.SemaphoreType.DMA((Y - 1,)),
            pltpu.SemaphoreType.DMA((Y - 1,)),
            pltpu.SemaphoreType.DMA((Y - 1,)),
            pltpu.SemaphoreType.DMA((Y - 1,)),
        ],
        compiler_params=pltpu.CompilerParams(collective_id=0),
    )(x, W)


# device time: 72025 ns/iter; 1.4667x vs baseline; 1.0848x over previous
import jax
import jax.numpy as jnp
from jax import lax
from jax.experimental import pallas as pl
from jax.experimental.pallas import tpu as pltpu

Y = 4


def kernel(x, W):
    t, d = x.shape
    _, v = W.shape
    quar = t // 4
    hq = quar // 2

    def body(x_ref, w_ref, out_ref, gather_ref,
             y_ss, y_rs,
             xq_ss, xq_rs,
             zq_ss, zq_rs,
             xr_ss, xr_rs,
             zr_ss, zr_rs):
        my_x = lax.axis_index("x")
        my_y = lax.axis_index("y")
        my_z = lax.axis_index("z")
        left = lax.rem(my_y + (Y - 1), Y)
        right = lax.rem(my_y + 1, Y)
        px = 1 - my_x
        pz = my_z + 1 - 2 * lax.rem(my_z, 2)
        bx = my_x
        bz = lax.rem(my_z, 2)
        ro_me = (2 * bx + bz) * quar
        ro_x = (2 * (1 - bx) + bz) * quar
        ro_z = (2 * bx + (1 - bz)) * quar
        ro_d = (2 * (1 - bx) + (1 - bz)) * quar

        barrier_sem = pltpu.get_barrier_semaphore()
        neighbors = [
            (my_x, left, my_z), (my_x, right, my_z),
            (px, my_y, my_z), (my_x, my_y, pz),
        ]
        for dev in neighbors:
            pl.semaphore_signal(
                barrier_sem, inc=1, device_id=dev,
                device_id_type=pl.DeviceIdType.MESH,
            )

        logits = jnp.dot(
            x_ref[...].astype(jnp.bfloat16),
            w_ref[...].astype(jnp.bfloat16),
            preferred_element_type=jnp.float32,
        )
        gather_ref[pl.ds(my_y, 1)] = logits.astype(jnp.bfloat16)[None]

        m = None
        s = None
        m_at = []
        slots = []

        def process(chunk_f32, slot, last):
            nonlocal m, s
            cm = chunk_f32.max(-1, keepdims=True)
            mk = cm if m is None else jnp.maximum(m, cm)
            e = jnp.exp(chunk_f32 - mk)
            es = e.sum(-1, keepdims=True)
            s = es if s is None else s * jnp.exp(m - mk) + es
            m = mk
            if last:
                out_ref[:, pl.ds(slot * v, v)] = e * (1.0 / s)
            else:
                out_ref[:, pl.ds(slot * v, v)] = e
                m_at.append(mk)
                slots.append(slot)

        process(logits, my_y, last=False)

        pl.semaphore_wait(barrier_sem, len(neighbors))

        def copy(slot, row0, nrows, sems_s, sems_r, h, dev):
            return pltpu.make_async_remote_copy(
                src_ref=gather_ref.at[slot, pl.ds(row0, nrows)],
                dst_ref=gather_ref.at[slot, pl.ds(row0, nrows)],
                send_sem=sems_s.at[h],
                recv_sem=sems_r.at[h],
                device_id=dev,
                device_id_type=pl.DeviceIdType.MESH,
            )

        to_x = (px, my_y, my_z)
        to_z = (my_x, my_y, pz)
        to_r = (my_x, right, my_z)

        waits = []
        quarters = {}
        relays = {}

        def piece_slot(h):
            return lax.rem(my_y + (Y - 1 - h), Y)

        for h in range(Y - 1):
            src_slot = my_y if h == 0 else lax.rem(my_y + (Y - h), Y)
            y_rdma = copy(src_slot, ro_me, quar, y_ss, y_rs, h, to_r)
            y_rdma.start()
            waits.append(y_rdma)

            if h >= 1:
                ps = piece_slot(h - 1)
                xq, zq = quarters[h - 1]
                zq.wait_recv()
                xr = copy(ps, ro_z, hq, xr_ss, xr_rs, h - 1, to_x)
                xr.start()
                xq.wait_recv()
                zr = copy(ps, ro_x + hq, hq, zr_ss, zr_rs, h - 1, to_z)
                zr.start()
                relays[h - 1] = (xr, zr)
                waits += [xr, zr]
            if h >= 2:
                ps = piece_slot(h - 2)
                xr, zr = relays[h - 2]
                xr.wait_recv()
                zr.wait_recv()
                process(
                    gather_ref[pl.ds(ps, 1)][0].astype(jnp.float32),
                    ps, last=False,
                )

            y_rdma.wait_recv()
            ps = piece_slot(h)
            xq = copy(ps, ro_me, quar, xq_ss, xq_rs, h, to_x)
            zq = copy(ps, ro_me, quar, zq_ss, zq_rs, h, to_z)
            xq.start()
            zq.start()
            quarters[h] = (xq, zq)
            waits += [xq, zq]

        ps2 = piece_slot(2)
        xq, zq = quarters[2]
        zq.wait_recv()
        xr = copy(ps2, ro_z, hq, xr_ss, xr_rs, 2, to_x)
        xr.start()
        xq.wait_recv()
        zr = copy(ps2, ro_x + hq, hq, zr_ss, zr_rs, 2, to_z)
        zr.start()
        waits += [xr, zr]

        ps1 = piece_slot(1)
        xr1, zr1 = relays[1]
        xr1.wait_recv()
        zr1.wait_recv()
        process(
            gather_ref[pl.ds(ps1, 1)][0].astype(jnp.float32),
            ps1, last=False,
        )

        xr.wait_recv()
        zr.wait_recv()
        process(
            gather_ref[pl.ds(ps2, 1)][0].astype(jnp.float32),
            ps2, last=True,
        )

        inv = 1.0 / s
        for mk, slot in zip(m_at, slots):
            factor = jnp.exp(mk - m) * inv
            out_ref[:, pl.ds(slot * v, v)] = (
                out_ref[:, pl.ds(slot * v, v)] * factor
            )

        for rdma in waits:
            rdma.wait_send()

        def exit_barrier(second_barrier):
            for dev in neighbors:
                pl.semaphore_signal(
                    second_barrier, inc=1, device_id=dev,
                    device_id_type=pl.DeviceIdType.MESH,
                )
            pl.semaphore_wait(second_barrier, len(neighbors))

        pl.run_scoped(exit_barrier, second_barrier=pltpu.SemaphoreType.REGULAR)

    dma = pltpu.SemaphoreType.DMA
    return pl.pallas_call(
        body,
        out_shape=jax.ShapeDtypeStruct((t, Y * v), jnp.float32),
        in_specs=[
            pl.BlockSpec(memory_space=pltpu.VMEM),
            pl.BlockSpec(memory_space=pltpu.VMEM),
        ],
        out_specs=pl.BlockSpec(memory_space=pltpu.VMEM),
        scratch_shapes=[pltpu.VMEM((Y, t, v), jnp.bfloat16)]
        + [dma((Y - 1,)) for _ in range(10)],
        compiler_params=pltpu.CompilerParams(collective_id=0),
    )(x, W)


# device time: 71451 ns/iter; 1.4785x vs baseline; 1.0080x over previous
import jax
import jax.numpy as jnp
from jax import lax
from jax.experimental import pallas as pl
from jax.experimental.pallas import tpu as pltpu

Y = 4


def kernel(x, W):
    t, d = x.shape
    _, v = W.shape
    quar = t // 4
    hq = quar // 2

    def body(x_ref, w_ref, out_ref, gather_ref, stats_ref,
             y_ss, y_rs,
             st_ss, st_rs,
             xq_ss, xq_rs,
             zq_ss, zq_rs,
             xr_ss, xr_rs,
             zr_ss, zr_rs):
        my_x = lax.axis_index("x")
        my_y = lax.axis_index("y")
        my_z = lax.axis_index("z")
        left = lax.rem(my_y + (Y - 1), Y)
        right = lax.rem(my_y + 1, Y)
        px = 1 - my_x
        pz = my_z + 1 - 2 * lax.rem(my_z, 2)
        bx = my_x
        bz = lax.rem(my_z, 2)
        ro_me = (2 * bx + bz) * quar
        ro_x = (2 * (1 - bx) + bz) * quar
        ro_z = (2 * bx + (1 - bz)) * quar

        barrier_sem = pltpu.get_barrier_semaphore()
        neighbors = [
            (my_x, left, my_z), (my_x, right, my_z),
            (px, my_y, my_z), (my_x, my_y, pz),
        ]
        for dev in neighbors:
            pl.semaphore_signal(
                barrier_sem, inc=1, device_id=dev,
                device_id_type=pl.DeviceIdType.MESH,
            )

        logits = jnp.dot(
            x_ref[...].astype(jnp.bfloat16),
            w_ref[...].astype(jnp.bfloat16),
            preferred_element_type=jnp.float32,
        )
        gather_ref[pl.ds(my_y, 1)] = logits.astype(jnp.bfloat16)[None]

        m_own = logits.max(-1, keepdims=True)
        e_own = jnp.exp(logits - m_own)
        s_own = e_own.sum(-1, keepdims=True)
        out_ref[:, pl.ds(my_y * v, v)] = e_own
        stats_ref[pl.ds(my_y, 1), :, 0:1] = m_own[None]
        stats_ref[pl.ds(my_y, 1), :, 1:2] = s_own[None]

        pl.semaphore_wait(barrier_sem, len(neighbors))

        def copy(slot, row0, nrows, sems_s, sems_r, h, dev):
            return pltpu.make_async_remote_copy(
                src_ref=gather_ref.at[slot, pl.ds(row0, nrows)],
                dst_ref=gather_ref.at[slot, pl.ds(row0, nrows)],
                send_sem=sems_s.at[h],
                recv_sem=sems_r.at[h],
                device_id=dev,
                device_id_type=pl.DeviceIdType.MESH,
            )

        def stats_copy(slot, h):
            return pltpu.make_async_remote_copy(
                src_ref=stats_ref.at[slot],
                dst_ref=stats_ref.at[slot],
                send_sem=st_ss.at[h],
                recv_sem=st_rs.at[h],
                device_id=(my_x, right, my_z),
                device_id_type=pl.DeviceIdType.MESH,
            )

        to_x = (px, my_y, my_z)
        to_z = (my_x, my_y, pz)
        to_r = (my_x, right, my_z)

        waits = []
        quarters = {}
        relays = {}
        st_rdmas = {}
        finals = {}

        def piece_slot(h):
            return lax.rem(my_y + (Y - 1 - h), Y)

        def process_final(slot):
            chunk = gather_ref[pl.ds(slot, 1)][0].astype(jnp.float32)
            e = jnp.exp(chunk - finals["m"])
            out_ref[:, pl.ds(slot * v, v)] = e * finals["inv"]

        for h in range(Y - 1):
            src_slot = my_y if h == 0 else lax.rem(my_y + (Y - h), Y)
            if h >= 1:
                st_rdmas[h - 1].wait_recv()
            st = stats_copy(src_slot, h)
            st.start()
            st_rdmas[h] = st
            waits.append(st)
            y_rdma = copy(src_slot, ro_me, quar, y_ss, y_rs, h, to_r)
            y_rdma.start()
            waits.append(y_rdma)

            if h >= 1:
                ps = piece_slot(h - 1)
                xq, zq = quarters[h - 1]
                zq.wait_recv()
                xr = copy(ps, ro_z, hq, xr_ss, xr_rs, h - 1, to_x)
                xr.start()
                xq.wait_recv()
                zr = copy(ps, ro_x + hq, hq, zr_ss, zr_rs, h - 1, to_z)
                zr.start()
                relays[h - 1] = (xr, zr)
                waits += [xr, zr]
            if h >= 2:
                st_rdmas[h].wait_recv()
                ms = stats_ref[:, :, 0:1]
                ss = stats_ref[:, :, 1:2]
                m_fin = jnp.max(ms, axis=0)
                s_fin = jnp.sum(ss * jnp.exp(ms - m_fin[None]), axis=0)
                finals["m"] = m_fin
                finals["inv"] = 1.0 / s_fin
                out_ref[:, pl.ds(my_y * v, v)] = (
                    out_ref[:, pl.ds(my_y * v, v)]
                    * (jnp.exp(m_own - m_fin) * finals["inv"])
                )
                xr, zr = relays[h - 2]
                xr.wait_recv()
                zr.wait_recv()
                process_final(piece_slot(h - 2))

            y_rdma.wait_recv()
            ps = piece_slot(h)
            xq = copy(ps, ro_me, quar, xq_ss, xq_rs, h, to_x)
            zq = copy(ps, ro_me, quar, zq_ss, zq_rs, h, to_z)
            xq.start()
            zq.start()
            quarters[h] = (xq, zq)
            waits += [xq, zq]

        ps2 = piece_slot(2)
        xq, zq = quarters[2]
        zq.wait_recv()
        xr = copy(ps2, ro_z, hq, xr_ss, xr_rs, 2, to_x)
        xr.start()
        xq.wait_recv()
        zr = copy(ps2, ro_x + hq, hq, zr_ss, zr_rs, 2, to_z)
        zr.start()
        waits += [xr, zr]

        xr1, zr1 = relays[1]
        xr1.wait_recv()
        zr1.wait_recv()
        process_final(piece_slot(1))

        xr.wait_recv()
        zr.wait_recv()
        process_final(ps2)

        for rdma in waits:
            rdma.wait_send()

    dma = pltpu.SemaphoreType.DMA
    return pl.pallas_call(
        body,
        out_shape=jax.ShapeDtypeStruct((t, Y * v), jnp.float32),
        in_specs=[
            pl.BlockSpec(memory_space=pltpu.VMEM),
            pl.BlockSpec(memory_space=pltpu.VMEM),
        ],
        out_specs=pl.BlockSpec(memory_space=pltpu.VMEM),
        scratch_shapes=[
            pltpu.VMEM((Y, t, v), jnp.bfloat16),
            pltpu.VMEM((Y, t, 2), jnp.float32),
        ]
        + [dma((Y - 1,)) for _ in range(12)],
        compiler_params=pltpu.CompilerParams(collective_id=0),
    )(x, W)


# device time: 67748 ns/iter; 1.5593x vs baseline; 1.0547x over previous
import jax
import jax.numpy as jnp
from jax import lax
from jax.experimental import pallas as pl
from jax.experimental.pallas import tpu as pltpu

Y = 4


def kernel(x, W):
    t, d = x.shape
    _, v = W.shape
    quar = t // 4
    hq = quar // 2

    def body(x_ref, w_ref, out_ref, gather_ref, stats_ref, stage_ref,
             cp_sems,
             y_ss, y_rs,
             st_ss, st_rs,
             xq_ss, xq_rs,
             zq_ss, zq_rs,
             xr_ss, xr_rs,
             zr_ss, zr_rs):
        my_x = lax.axis_index("x")
        my_y = lax.axis_index("y")
        my_z = lax.axis_index("z")
        left = lax.rem(my_y + (Y - 1), Y)
        right = lax.rem(my_y + 1, Y)
        px = 1 - my_x
        pz = my_z + 1 - 2 * lax.rem(my_z, 2)
        bx = my_x
        bz = lax.rem(my_z, 2)
        ro_me = (2 * bx + bz) * quar
        ro_x = (2 * (1 - bx) + bz) * quar
        ro_z = (2 * bx + (1 - bz)) * quar

        barrier_sem = pltpu.get_barrier_semaphore()
        neighbors = [
            (my_x, left, my_z), (my_x, right, my_z),
            (px, my_y, my_z), (my_x, my_y, pz),
        ]
        for dev in neighbors:
            pl.semaphore_signal(
                barrier_sem, inc=1, device_id=dev,
                device_id_type=pl.DeviceIdType.MESH,
            )

        logits = jnp.dot(
            x_ref[...].astype(jnp.bfloat16),
            w_ref[...].astype(jnp.bfloat16),
            preferred_element_type=jnp.float32,
        )
        gather_ref[pl.ds(my_y, 1)] = logits.astype(jnp.bfloat16)[None]

        m_own = logits.max(-1, keepdims=True)
        e_own = jnp.exp(logits - m_own)
        s_own = e_own.sum(-1, keepdims=True)
        stage_ref[pl.ds(my_y, 1)] = e_own[None]
        stats_ref[pl.ds(my_y, 1), :, 0:1] = m_own[None]
        stats_ref[pl.ds(my_y, 1), :, 1:2] = s_own[None]

        pl.semaphore_wait(barrier_sem, len(neighbors))

        def copy(slot, row0, nrows, sems_s, sems_r, h, dev):
            return pltpu.make_async_remote_copy(
                src_ref=gather_ref.at[slot, pl.ds(row0, nrows)],
                dst_ref=gather_ref.at[slot, pl.ds(row0, nrows)],
                send_sem=sems_s.at[h],
                recv_sem=sems_r.at[h],
                device_id=dev,
                device_id_type=pl.DeviceIdType.MESH,
            )

        def stats_copy(slot, h):
            return pltpu.make_async_remote_copy(
                src_ref=stats_ref.at[slot],
                dst_ref=stats_ref.at[slot],
                send_sem=st_ss.at[h],
                recv_sem=st_rs.at[h],
                device_id=(my_x, right, my_z),
                device_id_type=pl.DeviceIdType.MESH,
            )

        to_x = (px, my_y, my_z)
        to_z = (my_x, my_y, pz)
        to_r = (my_x, right, my_z)

        waits = []
        quarters = {}
        relays = {}
        st_rdmas = {}
        finals = {}

        def piece_slot(h):
            return lax.rem(my_y + (Y - 1 - h), Y)

        out_copies = []

        def flush(slot):
            cp = pltpu.make_async_copy(
                stage_ref.at[slot],
                out_ref.at[:, pl.ds(slot * v, v)],
                cp_sems.at[len(out_copies)],
            )
            cp.start()
            out_copies.append(cp)

        def process_final(slot):
            chunk = gather_ref[pl.ds(slot, 1)][0].astype(jnp.float32)
            e = jnp.exp(chunk - finals["m"])
            stage_ref[pl.ds(slot, 1)] = (e * finals["inv"])[None]
            flush(slot)

        for h in range(Y - 1):
            src_slot = my_y if h == 0 else lax.rem(my_y + (Y - h), Y)
            if h >= 1:
                st_rdmas[h - 1].wait_recv()
            st = stats_copy(src_slot, h)
            st.start()
            st_rdmas[h] = st
            waits.append(st)
            y_rdma = copy(src_slot, ro_me, quar, y_ss, y_rs, h, to_r)
            y_rdma.start()
            waits.append(y_rdma)

            if h >= 1:
                ps = piece_slot(h - 1)
                xq, zq = quarters[h - 1]
                zq.wait_recv()
                xr = copy(ps, ro_z, hq, xr_ss, xr_rs, h - 1, to_x)
                xr.start()
                xq.wait_recv()
                zr = copy(ps, ro_x + hq, hq, zr_ss, zr_rs, h - 1, to_z)
                zr.start()
                relays[h - 1] = (xr, zr)
                waits += [xr, zr]
            if h >= 2:
                st_rdmas[h].wait_recv()
                ms = stats_ref[:, :, 0:1]
                ss = stats_ref[:, :, 1:2]
                m_fin = jnp.max(ms, axis=0)
                s_fin = jnp.sum(ss * jnp.exp(ms - m_fin[None]), axis=0)
                finals["m"] = m_fin
                finals["inv"] = 1.0 / s_fin
                stage_ref[pl.ds(my_y, 1)] = (
                    stage_ref[pl.ds(my_y, 1)][0]
                    * (jnp.exp(m_own - m_fin) * finals["inv"])
                )[None]
                flush(my_y)
                xr, zr = relays[h - 2]
                xr.wait_recv()
                zr.wait_recv()
                process_final(piece_slot(h - 2))

            y_rdma.wait_recv()
            ps = piece_slot(h)
            xq = copy(ps, ro_me, quar, xq_ss, xq_rs, h, to_x)
            zq = copy(ps, ro_me, quar, zq_ss, zq_rs, h, to_z)
            xq.start()
            zq.start()
            quarters[h] = (xq, zq)
            waits += [xq, zq]

        ps2 = piece_slot(2)
        xq, zq = quarters[2]
        zq.wait_recv()
        xr = copy(ps2, ro_z, hq, xr_ss, xr_rs, 2, to_x)
        xr.start()
        xq.wait_recv()
        zr = copy(ps2, ro_x + hq, hq, zr_ss, zr_rs, 2, to_z)
        zr.start()
        waits += [xr, zr]

        xr1, zr1 = relays[1]
        xr1.wait_recv()
        zr1.wait_recv()
        process_final(piece_slot(1))

        xr.wait_recv()
        zr.wait_recv()
        process_final(ps2)

        for rdma in waits:
            rdma.wait_send()
        for cp in out_copies:
            cp.wait()

    dma = pltpu.SemaphoreType.DMA
    return pl.pallas_call(
        body,
        out_shape=jax.ShapeDtypeStruct((t, Y * v), jnp.float32),
        in_specs=[
            pl.BlockSpec(memory_space=pltpu.VMEM),
            pl.BlockSpec(memory_space=pltpu.VMEM),
        ],
        out_specs=pl.BlockSpec(memory_space=pl.ANY),
        scratch_shapes=[
            pltpu.VMEM((Y, t, v), jnp.bfloat16),
            pltpu.VMEM((Y, t, 2), jnp.float32),
            pltpu.VMEM((Y, t, v), jnp.float32),
            dma((Y,)),
        ]
        + [dma((Y - 1,)) for _ in range(12)],
        compiler_params=pltpu.CompilerParams(collective_id=0),
    )(x, W)


# device time: 62136 ns/iter; 1.7001x vs baseline; 1.0903x over previous
import jax
import jax.numpy as jnp
from jax import lax
from jax.experimental import pallas as pl
from jax.experimental.pallas import tpu as pltpu

Y = 4


def kernel(x, W):
    t, d = x.shape
    _, v = W.shape
    quar = t // 4
    hq = quar // 2

    def body(x_ref, w_ref, out_ref, gather_ref, stats_ref, stage_ref,
             cp_sems,
             y_ss, y_rs,
             st_ss, st_rs,
             xqa_ss, xqa_rs,
             xqb_ss, xqb_rs,
             zqa_ss, zqa_rs,
             zqb_ss, zqb_rs,
             xr_ss, xr_rs,
             zr_ss, zr_rs):
        my_x = lax.axis_index("x")
        my_y = lax.axis_index("y")
        my_z = lax.axis_index("z")
        left = lax.rem(my_y + (Y - 1), Y)
        right = lax.rem(my_y + 1, Y)
        px = 1 - my_x
        pz = my_z + 1 - 2 * lax.rem(my_z, 2)
        bx = my_x
        bz = lax.rem(my_z, 2)
        ro_me = (2 * bx + bz) * quar
        ro_x = (2 * (1 - bx) + bz) * quar
        ro_z = (2 * bx + (1 - bz)) * quar

        barrier_sem = pltpu.get_barrier_semaphore()
        neighbors = [
            (my_x, left, my_z), (my_x, right, my_z),
            (px, my_y, my_z), (my_x, my_y, pz),
        ]
        for dev in neighbors:
            pl.semaphore_signal(
                barrier_sem, inc=1, device_id=dev,
                device_id_type=pl.DeviceIdType.MESH,
            )

        logits = jnp.dot(
            x_ref[...].astype(jnp.bfloat16),
            w_ref[...].astype(jnp.bfloat16),
            preferred_element_type=jnp.float32,
        )
        gather_ref[pl.ds(my_y, 1)] = logits.astype(jnp.bfloat16)[None]

        m_own = logits.max(-1, keepdims=True)
        e_own = jnp.exp(logits - m_own)
        s_own = e_own.sum(-1, keepdims=True)
        stage_ref[pl.ds(my_y, 1)] = e_own[None]
        stats_ref[pl.ds(my_y, 1), :, 0:1] = m_own[None]
        stats_ref[pl.ds(my_y, 1), :, 1:2] = s_own[None]

        pl.semaphore_wait(barrier_sem, len(neighbors))

        def copy(slot, row0, nrows, sems_s, sems_r, h, dev):
            return pltpu.make_async_remote_copy(
                src_ref=gather_ref.at[slot, pl.ds(row0, nrows)],
                dst_ref=gather_ref.at[slot, pl.ds(row0, nrows)],
                send_sem=sems_s.at[h],
                recv_sem=sems_r.at[h],
                device_id=dev,
                device_id_type=pl.DeviceIdType.MESH,
            )

        def stats_copy(slot, h):
            return pltpu.make_async_remote_copy(
                src_ref=stats_ref.at[slot],
                dst_ref=stats_ref.at[slot],
                send_sem=st_ss.at[h],
                recv_sem=st_rs.at[h],
                device_id=(my_x, right, my_z),
                device_id_type=pl.DeviceIdType.MESH,
            )

        to_x = (px, my_y, my_z)
        to_z = (my_x, my_y, pz)
        to_r = (my_x, right, my_z)

        waits = []
        quarters = {}
        relays = {}
        st_rdmas = {}
        finals = {}

        def piece_slot(h):
            return lax.rem(my_y + (Y - 1 - h), Y)

        out_copies = []

        def flush(slot):
            cp = pltpu.make_async_copy(
                stage_ref.at[slot],
                out_ref.at[:, pl.ds(slot * v, v)],
                cp_sems.at[len(out_copies)],
            )
            cp.start()
            out_copies.append(cp)

        def process_final(slot):
            chunk = gather_ref[pl.ds(slot, 1)][0].astype(jnp.float32)
            e = jnp.exp(chunk - finals["m"])
            stage_ref[pl.ds(slot, 1)] = (e * finals["inv"])[None]
            flush(slot)

        def issue_relays(k):
            ps = piece_slot(k)
            xqa, xqb, zqa, zqb = quarters[k]
            zqa.wait_recv()
            xr = copy(ps, ro_z, hq, xr_ss, xr_rs, k, to_x)
            xr.start()
            xqb.wait_recv()
            zr = copy(ps, ro_x + hq, hq, zr_ss, zr_rs, k, to_z)
            zr.start()
            relays[k] = (xr, zr)
            waits.extend([xr, zr])

        def complete_piece(k, last=False):
            xqa, xqb, zqa, zqb = quarters[k]
            xqa.wait_recv()
            zqb.wait_recv()
            xr, zr = relays[k]
            xr.wait_recv()
            zr.wait_recv()
            process_final(piece_slot(k))

        for h in range(Y - 1):
            src_slot = my_y if h == 0 else lax.rem(my_y + (Y - h), Y)
            if h >= 1:
                st_rdmas[h - 1].wait_recv()
            st = stats_copy(src_slot, h)
            st.start()
            st_rdmas[h] = st
            waits.append(st)
            y_rdma = copy(src_slot, ro_me, quar, y_ss, y_rs, h, to_r)
            y_rdma.start()
            waits.append(y_rdma)

            if h >= 1:
                issue_relays(h - 1)
            if h >= 2:
                st_rdmas[h].wait_recv()
                ms = stats_ref[:, :, 0:1]
                ss = stats_ref[:, :, 1:2]
                m_fin = jnp.max(ms, axis=0)
                s_fin = jnp.sum(ss * jnp.exp(ms - m_fin[None]), axis=0)
                finals["m"] = m_fin
                finals["inv"] = 1.0 / s_fin
                stage_ref[pl.ds(my_y, 1)] = (
                    stage_ref[pl.ds(my_y, 1)][0]
                    * (jnp.exp(m_own - m_fin) * finals["inv"])
                )[None]
                flush(my_y)
                complete_piece(h - 2)

            y_rdma.wait_recv()
            ps = piece_slot(h)
            xqb = copy(ps, ro_me + hq, hq, xqb_ss, xqb_rs, h, to_x)
            xqa = copy(ps, ro_me, hq, xqa_ss, xqa_rs, h, to_x)
            zqa = copy(ps, ro_me, hq, zqa_ss, zqa_rs, h, to_z)
            zqb = copy(ps, ro_me + hq, hq, zqb_ss, zqb_rs, h, to_z)
            xqb.start()
            xqa.start()
            zqa.start()
            zqb.start()
            quarters[h] = (xqa, xqb, zqa, zqb)
            waits.extend([xqa, xqb, zqa, zqb])

        issue_relays(2)
        complete_piece(1)
        complete_piece(2, last=True)

        for rdma in waits:
            rdma.wait_send()
        for cp in out_copies:
            cp.wait()

    dma = pltpu.SemaphoreType.DMA
    return pl.pallas_call(
        body,
        out_shape=jax.ShapeDtypeStruct((t, Y * v), jnp.float32),
        in_specs=[
            pl.BlockSpec(memory_space=pltpu.VMEM),
            pl.BlockSpec(memory_space=pltpu.VMEM),
        ],
        out_specs=pl.BlockSpec(memory_space=pl.ANY),
        scratch_shapes=[
            pltpu.VMEM((Y, t, v), jnp.bfloat16),
            pltpu.VMEM((Y, t, 2), jnp.float32),
            pltpu.VMEM((Y, t, v), jnp.float32),
            dma((Y,)),
        ]
        + [dma((Y - 1,)) for _ in range(16)],
        compiler_params=pltpu.CompilerParams(collective_id=0),
    )(x, W)
